# baseline (speedup 1.0000x reference)
import jax
import jax.numpy as jnp
from jax.experimental import pallas as pl

_TOPK = 13
_EPS = 1e-9


def _dot_exact(a, b, dims, split_lhs=False):
    x = a if split_lhs else b
    hi = x.astype(jnp.bfloat16).astype(jnp.float32)
    out = None
    for term in (hi, x - hi):
        lhs, rhs = (term, b) if split_lhs else (a, term)
        part = jax.lax.dot_general(lhs, rhs, (dims, ((), ())),
                                   preferred_element_type=jnp.float32,
                                   precision=jax.lax.Precision.DEFAULT)
        out = part if out is None else out + part
    return out


def _assigner_body(ps_ref, pb_ref, apt_ref, gl_ref, gb_ref, mg_ref,
                   tb_ref, ts_ref, fg_ref):
    f32 = jnp.float32
    ps = ps_ref[0]
    pb = pb_ref[0]
    pbt = jnp.transpose(pb)
    apt = apt_ref[...]
    gl = gl_ref[0]
    gb = gb_ref[0]
    mg = mg_ref[0]
    n = gb.shape[0]
    A = ps.shape[0]
    nc = ps.shape[1]

    ax = apt[0:1, :]
    ay = apt[1:2, :]
    gx1 = gb[:, 0:1]
    gy1 = gb[:, 1:2]
    gx2 = gb[:, 2:3]
    gy2 = gb[:, 3:4]

    mind = jnp.minimum(jnp.minimum(ax - gx1, ay - gy1),
                       jnp.minimum(gx2 - ax, gy2 - ay))
    mask3 = (mind > 1e-9) & (mg > 0.0)

    lab_oh = (gl == jax.lax.broadcasted_iota(jnp.int32, (n, nc), 1)).astype(f32)
    bscore = _dot_exact(lab_oh, ps, ((1,), (1,)))
    bscore = jnp.where(mask3, bscore, 0.0)

    px1 = pbt[0:1, :]
    py1 = pbt[1:2, :]
    px2 = pbt[2:3, :]
    py2 = pbt[3:4, :]
    iw = jnp.maximum(jnp.minimum(gx2, px2) - jnp.maximum(gx1, px1), 0.0)
    ih = jnp.maximum(jnp.minimum(gy2, py2) - jnp.maximum(gy1, py1), 0.0)
    inter = iw * ih
    area_g = (gx2 - gx1) * (gy2 - gy1)
    area_p = (px2 - px1) * (py2 - py1)
    iou = inter / (area_g + area_p - inter + 1e-7)
    ov = jnp.where(mask3, jnp.maximum(iou, 0.0), 0.0)

    o2 = ov * ov
    align = bscore * (o2 * o2 * o2)

    iota_af = jax.lax.broadcasted_iota(jnp.int32, (n, A), 1).astype(f32)
    sent = jnp.float32(-3e38)
    key = jnp.where(align > 0.0, align, -1.0 - iota_af)
    for _ in range(_TOPK):
        m = jnp.max(key, axis=1, keepdims=True)
        key = jnp.where(key == m, sent, key)
    sel = key == sent

    mp = (sel & mask3).astype(f32)
    fg = jnp.sum(mp, axis=0, keepdims=True)

    iota_n = jax.lax.broadcasted_iota(jnp.int32, (n, A), 0)
    omax = jnp.max(ov, axis=0, keepdims=True)
    firstn = jnp.min(jnp.where(ov == omax, iota_n, n), axis=0, keepdims=True)
    ismax = (iota_n == firstn).astype(f32)
    mp = jnp.where(fg > 1.0, ismax, mp)
    fg = jnp.sum(mp, axis=0, keepdims=True)

    fidx = jnp.min(jnp.where(mp > 0.5, iota_n, n), axis=0, keepdims=True)
    tgi = jnp.where(fg > 0.0, fidx, 0)
    oh_t = (iota_n == tgi).astype(f32)

    am = align * mp
    pos_align = jnp.max(am, axis=1, keepdims=True)
    pos_ov = jnp.max(ov * mp, axis=1, keepdims=True)
    norm = jnp.max(am * (pos_ov / (pos_align + _EPS)),
                   axis=0, keepdims=True)

    tb_ref[0] = _dot_exact(oh_t, gb, ((0,), (0,)))
    w = oh_t * jnp.where(fg > 0.0, norm, 0.0)
    ts_ref[0] = _dot_exact(w, lab_oh, ((0,), (0,)),
                           split_lhs=True)
    fg_ref[0] = fg


@jax.jit
def kernel(pred_scores, pred_bboxes, anchor_points, gt_labels, gt_bboxes,
           mask_gt):
    bs, A, nc = pred_scores.shape
    n = gt_bboxes.shape[1]
    apt = jnp.transpose(anchor_points, (1, 0))
    gl = gt_labels.astype(jnp.int32)
    mg = mask_gt.astype(jnp.float32)

    tb, ts, fg = pl.pallas_call(
        _assigner_body,
        grid=(bs,),
        in_specs=[
            pl.BlockSpec((1, A, nc), lambda b: (b, 0, 0)),
            pl.BlockSpec((1, A, 4), lambda b: (b, 0, 0)),
            pl.BlockSpec((2, A), lambda b: (0, 0)),
            pl.BlockSpec((1, n, 1), lambda b: (b, 0, 0)),
            pl.BlockSpec((1, n, 4), lambda b: (b, 0, 0)),
            pl.BlockSpec((1, n, 1), lambda b: (b, 0, 0)),
        ],
        out_specs=[
            pl.BlockSpec((1, A, 4), lambda b: (b, 0, 0)),
            pl.BlockSpec((1, A, nc), lambda b: (b, 0, 0)),
            pl.BlockSpec((1, 1, A), lambda b: (b, 0, 0)),
        ],
        out_shape=[
            jax.ShapeDtypeStruct((bs, A, 4), jnp.float32),
            jax.ShapeDtypeStruct((bs, A, nc), jnp.float32),
            jax.ShapeDtypeStruct((bs, 1, A), jnp.float32),
        ],
    )(pred_scores.astype(jnp.float32), pred_bboxes.astype(jnp.float32),
      apt, gl, gt_bboxes.astype(jnp.float32), mg)

    return tb, ts, fg.reshape(bs, A) > 0.0


# speedup vs baseline: 1.3229x; 1.3229x over previous
import jax
import jax.numpy as jnp
from jax.experimental import pallas as pl

_TOPK = 13
_EPS = 1e-9


def _dot_exact(a, b, dims, split_lhs=False):
    x = a if split_lhs else b
    hi = x.astype(jnp.bfloat16).astype(jnp.float32)
    out = None
    for term in (hi, x - hi):
        lhs, rhs = (term, b) if split_lhs else (a, term)
        part = jax.lax.dot_general(lhs, rhs, (dims, ((), ())),
                                   preferred_element_type=jnp.float32,
                                   precision=jax.lax.Precision.DEFAULT)
        out = part if out is None else out + part
    return out


def _assigner_body(ps_ref, pbt_ref, apt_ref, gl_ref, gb_ref, mg_ref,
                   tb_ref, ts_ref, fg_ref):
    f32 = jnp.float32
    ps = ps_ref[0]
    pbt = pbt_ref[0]
    apt = apt_ref[...]
    gl = gl_ref[0]
    gb = gb_ref[0]
    mg = mg_ref[0]
    n = gb.shape[0]
    A = ps.shape[0]
    nc = ps.shape[1]

    ax = apt[0:1, :]
    ay = apt[1:2, :]
    gx1 = gb[:, 0:1]
    gy1 = gb[:, 1:2]
    gx2 = gb[:, 2:3]
    gy2 = gb[:, 3:4]

    mind = jnp.minimum(jnp.minimum(ax - gx1, ay - gy1),
                       jnp.minimum(gx2 - ax, gy2 - ay))
    mask3 = (mind > 1e-9) & (mg > 0.0)

    lab_oh = (gl == jax.lax.broadcasted_iota(jnp.int32, (n, nc), 1)).astype(f32)
    bscore = _dot_exact(lab_oh, ps, ((1,), (1,)))
    bscore = jnp.where(mask3, bscore, 0.0)

    px1 = pbt[0:1, :]
    py1 = pbt[1:2, :]
    px2 = pbt[2:3, :]
    py2 = pbt[3:4, :]
    iw = jnp.maximum(jnp.minimum(gx2, px2) - jnp.maximum(gx1, px1), 0.0)
    ih = jnp.maximum(jnp.minimum(gy2, py2) - jnp.maximum(gy1, py1), 0.0)
    inter = iw * ih
    area_g = (gx2 - gx1) * (gy2 - gy1)
    area_p = (px2 - px1) * (py2 - py1)
    iou = inter / (area_g + area_p - inter + 1e-7)
    ov = jnp.where(mask3, jnp.maximum(iou, 0.0), 0.0)

    o2 = ov * ov
    align = bscore * (o2 * o2 * o2)

    iota_af = jax.lax.broadcasted_iota(jnp.int32, (n, A), 1).astype(f32)
    sent = jnp.float32(-3e38)
    key = jnp.where(align > 0.0, align, -1.0 - iota_af)
    for _ in range(_TOPK):
        m = jnp.max(key, axis=1, keepdims=True)
        key = jnp.where(key == m, sent, key)
    sel = key == sent

    mp = (sel & mask3).astype(f32)
    fg = jnp.sum(mp, axis=0, keepdims=True)

    iota_n = jax.lax.broadcasted_iota(jnp.int32, (n, A), 0)
    omax = jnp.max(ov, axis=0, keepdims=True)
    firstn = jnp.min(jnp.where(ov == omax, iota_n, n), axis=0, keepdims=True)
    ismax = (iota_n == firstn).astype(f32)
    mp = jnp.where(fg > 1.0, ismax, mp)
    fg = jnp.sum(mp, axis=0, keepdims=True)

    fidx = jnp.min(jnp.where(mp > 0.5, iota_n, n), axis=0, keepdims=True)
    tgi = jnp.where(fg > 0.0, fidx, 0)
    oh_t = (iota_n == tgi).astype(f32)

    am = align * mp
    pos_align = jnp.max(am, axis=1, keepdims=True)
    pos_ov = jnp.max(ov * mp, axis=1, keepdims=True)
    norm = jnp.max(am * (pos_ov / (pos_align + _EPS)),
                   axis=0, keepdims=True)

    tb_ref[0] = _dot_exact(gb, oh_t, ((0,), (0,)),
                           split_lhs=True)
    w = oh_t * jnp.where(fg > 0.0, norm, 0.0)
    ts_ref[0] = _dot_exact(w, lab_oh, ((0,), (0,)),
                           split_lhs=True)
    fg_ref[0] = fg


@jax.jit
def kernel(pred_scores, pred_bboxes, anchor_points, gt_labels, gt_bboxes,
           mask_gt):
    bs, A, nc = pred_scores.shape
    n = gt_bboxes.shape[1]
    pbt = jnp.transpose(pred_bboxes, (0, 2, 1))
    apt = jnp.transpose(anchor_points, (1, 0))
    gl = gt_labels.astype(jnp.int32)
    mg = mask_gt.astype(jnp.float32)

    tb, ts, fg = pl.pallas_call(
        _assigner_body,
        grid=(bs,),
        in_specs=[
            pl.BlockSpec((1, A, nc), lambda b: (b, 0, 0)),
            pl.BlockSpec((1, 4, A), lambda b: (b, 0, 0)),
            pl.BlockSpec((2, A), lambda b: (0, 0)),
            pl.BlockSpec((1, n, 1), lambda b: (b, 0, 0)),
            pl.BlockSpec((1, n, 4), lambda b: (b, 0, 0)),
            pl.BlockSpec((1, n, 1), lambda b: (b, 0, 0)),
        ],
        out_specs=[
            pl.BlockSpec((1, 4, A), lambda b: (b, 0, 0)),
            pl.BlockSpec((1, A, nc), lambda b: (b, 0, 0)),
            pl.BlockSpec((1, 1, A), lambda b: (b, 0, 0)),
        ],
        out_shape=[
            jax.ShapeDtypeStruct((bs, 4, A), jnp.float32),
            jax.ShapeDtypeStruct((bs, A, nc), jnp.float32),
            jax.ShapeDtypeStruct((bs, 1, A), jnp.float32),
        ],
    )(pred_scores.astype(jnp.float32), pbt, apt, gl,
      gt_bboxes.astype(jnp.float32), mg)

    return jnp.transpose(tb, (0, 2, 1)), ts, fg.reshape(bs, A) > 0.0


# speedup vs baseline: 1.9043x; 1.4395x over previous
import jax
import jax.numpy as jnp
from jax.experimental import pallas as pl

_TOPK = 13
_EPS = 1e-9


def _dot_exact(a, b, dims, split_lhs=False):
    x = a if split_lhs else b
    hi = x.astype(jnp.bfloat16).astype(jnp.float32)
    out = None
    for term in (hi, x - hi):
        lhs, rhs = (term, b) if split_lhs else (a, term)
        part = jax.lax.dot_general(lhs, rhs, (dims, ((), ())),
                                   preferred_element_type=jnp.float32,
                                   precision=jax.lax.Precision.DEFAULT)
        out = part if out is None else out + part
    return out


def _assigner_body(ps_ref, pbt_ref, apt_ref, gl_ref, gb_ref, mg_ref,
                   tb_ref, ts_ref, fg_ref):
    f32 = jnp.float32
    ps = ps_ref[0]
    pbt = pbt_ref[0]
    apt = apt_ref[...]
    gl = gl_ref[0]
    gb = gb_ref[0]
    mg = mg_ref[0]
    n = gb.shape[0]
    A = ps.shape[0]
    nc = ps.shape[1]

    ax = apt[0:1, :]
    ay = apt[1:2, :]
    gx1 = gb[:, 0:1]
    gy1 = gb[:, 1:2]
    gx2 = gb[:, 2:3]
    gy2 = gb[:, 3:4]

    mind = jnp.minimum(jnp.minimum(ax - gx1, ay - gy1),
                       jnp.minimum(gx2 - ax, gy2 - ay))
    mask3 = (mind > 1e-9) & (mg > 0.0)

    lab_oh = (gl == jax.lax.broadcasted_iota(jnp.int32, (n, nc), 1)).astype(f32)
    bscore = _dot_exact(lab_oh, ps, ((1,), (1,)))
    bscore = jnp.where(mask3, bscore, 0.0)

    px1 = pbt[0:1, :]
    py1 = pbt[1:2, :]
    px2 = pbt[2:3, :]
    py2 = pbt[3:4, :]
    iw = jnp.maximum(jnp.minimum(gx2, px2) - jnp.maximum(gx1, px1), 0.0)
    ih = jnp.maximum(jnp.minimum(gy2, py2) - jnp.maximum(gy1, py1), 0.0)
    inter = iw * ih
    area_g = (gx2 - gx1) * (gy2 - gy1)
    area_p = (px2 - px1) * (py2 - py1)
    iou = inter / (area_g + area_p - inter + 1e-7)
    ov = jnp.where(mask3, jnp.maximum(iou, 0.0), 0.0)

    o2 = ov * ov
    align = bscore * (o2 * o2 * o2)

    iota_af = jax.lax.broadcasted_iota(jnp.int32, (n, A), 1).astype(f32)
    sent = jnp.float32(-3e38)
    key = jnp.where(align > 0.0, align, -1.0 - iota_af)
    for _ in range(_TOPK):
        m = jnp.max(key, axis=1, keepdims=True)
        key = jnp.where(key == m, sent, key)
    sel = key == sent

    mp = (sel & mask3).astype(f32)
    fg = jnp.sum(mp, axis=0, keepdims=True)

    iota_n = jax.lax.broadcasted_iota(jnp.int32, (n, A), 0)
    omax = jnp.max(ov, axis=0, keepdims=True)
    firstn = jnp.min(jnp.where(ov == omax, iota_n, n), axis=0, keepdims=True)
    ismax = (iota_n == firstn).astype(f32)
    mp = jnp.where(fg > 1.0, ismax, mp)
    fg = jnp.sum(mp, axis=0, keepdims=True)

    fidx = jnp.min(jnp.where(mp > 0.5, iota_n, n), axis=0, keepdims=True)
    tgi = jnp.where(fg > 0.0, fidx, 0)
    oh_t = (iota_n == tgi).astype(f32)

    am = align * mp
    pos_align = jnp.max(am, axis=1, keepdims=True)
    pos_ov = jnp.max(ov * mp, axis=1, keepdims=True)
    norm = jnp.max(am * (pos_ov / (pos_align + _EPS)),
                   axis=0, keepdims=True)

    tb_ref[0] = _dot_exact(gb, oh_t, ((0,), (0,)),
                           split_lhs=True)
    w = oh_t * jnp.where(fg > 0.0, norm, 0.0)
    ts_ref[0] = _dot_exact(lab_oh, w, ((0,), (0,)))
    fg_ref[0] = fg


@jax.jit
def kernel(pred_scores, pred_bboxes, anchor_points, gt_labels, gt_bboxes,
           mask_gt):
    bs, A, nc = pred_scores.shape
    n = gt_bboxes.shape[1]
    pbt = jnp.transpose(pred_bboxes, (0, 2, 1))
    apt = jnp.transpose(anchor_points, (1, 0))
    gl = gt_labels.astype(jnp.int32)
    mg = mask_gt.astype(jnp.float32)

    tb, ts, fg = pl.pallas_call(
        _assigner_body,
        grid=(bs,),
        in_specs=[
            pl.BlockSpec((1, A, nc), lambda b: (b, 0, 0)),
            pl.BlockSpec((1, 4, A), lambda b: (b, 0, 0)),
            pl.BlockSpec((2, A), lambda b: (0, 0)),
            pl.BlockSpec((1, n, 1), lambda b: (b, 0, 0)),
            pl.BlockSpec((1, n, 4), lambda b: (b, 0, 0)),
            pl.BlockSpec((1, n, 1), lambda b: (b, 0, 0)),
        ],
        out_specs=[
            pl.BlockSpec((1, 4, A), lambda b: (b, 0, 0)),
            pl.BlockSpec((1, nc, A), lambda b: (b, 0, 0)),
            pl.BlockSpec((1, 1, A), lambda b: (b, 0, 0)),
        ],
        out_shape=[
            jax.ShapeDtypeStruct((bs, 4, A), jnp.float32),
            jax.ShapeDtypeStruct((bs, nc, A), jnp.float32),
            jax.ShapeDtypeStruct((bs, 1, A), jnp.float32),
        ],
    )(pred_scores.astype(jnp.float32), pbt, apt, gl,
      gt_bboxes.astype(jnp.float32), mg)

    return (jnp.transpose(tb, (0, 2, 1)), jnp.transpose(ts, (0, 2, 1)),
            fg.reshape(bs, A) > 0.0)


# speedup vs baseline: 2.9975x; 1.5741x over previous
import jax
import jax.numpy as jnp
from jax.experimental import pallas as pl

_TOPK = 13
_EPS = 1e-9


def _dot_exact(a, b, dims, split_lhs=False):
    x = a if split_lhs else b
    hi = x.astype(jnp.bfloat16).astype(jnp.float32)
    out = None
    for term in (hi, x - hi):
        lhs, rhs = (term, b) if split_lhs else (a, term)
        part = jax.lax.dot_general(lhs, rhs, (dims, ((), ())),
                                   preferred_element_type=jnp.float32,
                                   precision=jax.lax.Precision.DEFAULT)
        out = part if out is None else out + part
    return out


def _assigner_body(ps_ref, pbt_ref, apt_ref, gl_ref, gb_ref, mg_ref,
                   tb_ref, ts_ref, fg_ref):
    f32 = jnp.float32
    ps = ps_ref[0]
    pbt = pbt_ref[0]
    apt = apt_ref[...]
    gl = gl_ref[0]
    gb = gb_ref[0]
    mg = mg_ref[0]
    n = gb.shape[0]
    nc, A = ps.shape

    ax = apt[0:1, :]
    ay = apt[1:2, :]
    gx1 = gb[:, 0:1]
    gy1 = gb[:, 1:2]
    gx2 = gb[:, 2:3]
    gy2 = gb[:, 3:4]

    mind = jnp.minimum(jnp.minimum(ax - gx1, ay - gy1),
                       jnp.minimum(gx2 - ax, gy2 - ay))
    mask3 = (mind > 1e-9) & (mg > 0.0)

    lab_oh = (gl == jax.lax.broadcasted_iota(jnp.int32, (n, nc), 1)).astype(f32)
    bscore = _dot_exact(lab_oh, ps, ((1,), (0,)))
    bscore = jnp.where(mask3, bscore, 0.0)

    px1 = pbt[0:1, :]
    py1 = pbt[1:2, :]
    px2 = pbt[2:3, :]
    py2 = pbt[3:4, :]
    iw = jnp.maximum(jnp.minimum(gx2, px2) - jnp.maximum(gx1, px1), 0.0)
    ih = jnp.maximum(jnp.minimum(gy2, py2) - jnp.maximum(gy1, py1), 0.0)
    inter = iw * ih
    area_g = (gx2 - gx1) * (gy2 - gy1)
    area_p = (px2 - px1) * (py2 - py1)
    iou = inter / (area_g + area_p - inter + 1e-7)
    ov = jnp.where(mask3, jnp.maximum(iou, 0.0), 0.0)

    o2 = ov * ov
    align = bscore * (o2 * o2 * o2)

    iota_af = jax.lax.broadcasted_iota(jnp.int32, (n, A), 1).astype(f32)
    sent = jnp.float32(-3e38)
    key = jnp.where(align > 0.0, align, -1.0 - iota_af)
    for _ in range(_TOPK):
        m = jnp.max(key, axis=1, keepdims=True)
        key = jnp.where(key == m, sent, key)
    sel = key == sent

    mp = (sel & mask3).astype(f32)
    fg = jnp.sum(mp, axis=0, keepdims=True)

    iota_n = jax.lax.broadcasted_iota(jnp.int32, (n, A), 0)
    omax = jnp.max(ov, axis=0, keepdims=True)
    firstn = jnp.min(jnp.where(ov == omax, iota_n, n), axis=0, keepdims=True)
    ismax = (iota_n == firstn).astype(f32)
    mp = jnp.where(fg > 1.0, ismax, mp)
    fg = jnp.sum(mp, axis=0, keepdims=True)

    fidx = jnp.min(jnp.where(mp > 0.5, iota_n, n), axis=0, keepdims=True)
    tgi = jnp.where(fg > 0.0, fidx, 0)
    oh_t = (iota_n == tgi).astype(f32)

    am = align * mp
    pos_align = jnp.max(am, axis=1, keepdims=True)
    pos_ov = jnp.max(ov * mp, axis=1, keepdims=True)
    norm = jnp.max(am * (pos_ov / (pos_align + _EPS)),
                   axis=0, keepdims=True)

    tb_ref[0] = _dot_exact(gb, oh_t, ((0,), (0,)),
                           split_lhs=True)
    w = oh_t * jnp.where(fg > 0.0, norm, 0.0)
    ts_ref[0] = _dot_exact(lab_oh, w, ((0,), (0,)))
    fg_ref[0] = fg


@jax.jit
def kernel(pred_scores, pred_bboxes, anchor_points, gt_labels, gt_bboxes,
           mask_gt):
    bs, A, nc = pred_scores.shape
    n = gt_bboxes.shape[1]
    pst = jnp.transpose(pred_scores, (0, 2, 1))
    pbt = jnp.transpose(pred_bboxes, (0, 2, 1))
    apt = jnp.transpose(anchor_points, (1, 0))
    gl = gt_labels.astype(jnp.int32)
    mg = mask_gt.astype(jnp.float32)

    tb, ts, fg = pl.pallas_call(
        _assigner_body,
        grid=(bs,),
        in_specs=[
            pl.BlockSpec((1, nc, A), lambda b: (b, 0, 0)),
            pl.BlockSpec((1, 4, A), lambda b: (b, 0, 0)),
            pl.BlockSpec((2, A), lambda b: (0, 0)),
            pl.BlockSpec((1, n, 1), lambda b: (b, 0, 0)),
            pl.BlockSpec((1, n, 4), lambda b: (b, 0, 0)),
            pl.BlockSpec((1, n, 1), lambda b: (b, 0, 0)),
        ],
        out_specs=[
            pl.BlockSpec((1, 4, A), lambda b: (b, 0, 0)),
            pl.BlockSpec((1, nc, A), lambda b: (b, 0, 0)),
            pl.BlockSpec((1, 1, A), lambda b: (b, 0, 0)),
        ],
        out_shape=[
            jax.ShapeDtypeStruct((bs, 4, A), jnp.float32),
            jax.ShapeDtypeStruct((bs, nc, A), jnp.float32),
            jax.ShapeDtypeStruct((bs, 1, A), jnp.float32),
        ],
    )(pst.astype(jnp.float32), pbt, apt, gl,
      gt_bboxes.astype(jnp.float32), mg)

    return (jnp.transpose(tb, (0, 2, 1)), jnp.transpose(ts, (0, 2, 1)),
            fg.reshape(bs, A) > 0.0)


# speedup vs baseline: 3.3329x; 1.1119x over previous
import jax
import jax.numpy as jnp
from jax.experimental import pallas as pl

_TOPK = 13
_EPS = 1e-9


def _dot_exact(a, b, dims, split_lhs=False):
    x = a if split_lhs else b
    hi = x.astype(jnp.bfloat16).astype(jnp.float32)
    out = None
    for term in (hi, x - hi):
        lhs, rhs = (term, b) if split_lhs else (a, term)
        part = jax.lax.dot_general(lhs, rhs, (dims, ((), ())),
                                   preferred_element_type=jnp.float32,
                                   precision=jax.lax.Precision.DEFAULT)
        out = part if out is None else out + part
    return out


def _assigner_body(ps_ref, pbt_ref, apt_ref, gl_ref, gb_ref, mg_ref,
                   tb_ref, ts_ref, fg_ref):
    f32 = jnp.float32
    ps = ps_ref[0]
    pbt = pbt_ref[0]
    apt = apt_ref[...]
    gl = gl_ref[0]
    gb = gb_ref[0]
    mg = mg_ref[0]
    n = gb.shape[0]
    nc, A = ps.shape

    ax = apt[0:1, :]
    ay = apt[1:2, :]
    gx1 = gb[:, 0:1]
    gy1 = gb[:, 1:2]
    gx2 = gb[:, 2:3]
    gy2 = gb[:, 3:4]

    mind = jnp.minimum(jnp.minimum(ax - gx1, ay - gy1),
                       jnp.minimum(gx2 - ax, gy2 - ay))
    mask3 = (mind > 1e-9) & (mg > 0.0)

    lab_oh = (gl == jax.lax.broadcasted_iota(jnp.int32, (n, nc), 1)).astype(f32)
    bscore = _dot_exact(lab_oh, ps, ((1,), (0,)))
    bscore = jnp.where(mask3, bscore, 0.0)

    px1 = pbt[0:1, :]
    py1 = pbt[1:2, :]
    px2 = pbt[2:3, :]
    py2 = pbt[3:4, :]
    iw = jnp.maximum(jnp.minimum(gx2, px2) - jnp.maximum(gx1, px1), 0.0)
    ih = jnp.maximum(jnp.minimum(gy2, py2) - jnp.maximum(gy1, py1), 0.0)
    inter = iw * ih
    area_g = (gx2 - gx1) * (gy2 - gy1)
    area_p = (px2 - px1) * (py2 - py1)
    iou = inter / (area_g + area_p - inter + 1e-7)
    ov = jnp.where(mask3, jnp.maximum(iou, 0.0), 0.0)

    o2 = ov * ov
    align = bscore * (o2 * o2 * o2)

    iota_af = jax.lax.broadcasted_iota(jnp.int32, (n, A), 1).astype(f32)
    sent = jnp.float32(-3e38)
    key = jnp.where(align > 0.0, align, -1.0 - iota_af)
    for _ in range(_TOPK):
        m = jnp.max(key, axis=1, keepdims=True)
        key = jnp.where(key == m, sent, key)
    sel = key == sent

    mp = (sel & mask3).astype(f32)
    fg = jnp.sum(mp, axis=0, keepdims=True)

    iota_n = jax.lax.broadcasted_iota(jnp.int32, (n, A), 0)
    omax = jnp.max(ov, axis=0, keepdims=True)
    firstn = jnp.min(jnp.where(ov == omax, iota_n, n), axis=0, keepdims=True)
    ismax = (iota_n == firstn).astype(f32)
    mp = jnp.where(fg > 1.0, ismax, mp)
    fg = jnp.sum(mp, axis=0, keepdims=True)

    fidx = jnp.min(jnp.where(mp > 0.5, iota_n, n), axis=0, keepdims=True)
    tgi = jnp.where(fg > 0.0, fidx, 0)
    oh_t = (iota_n == tgi).astype(f32)

    am = align * mp
    pos_align = jnp.max(am, axis=1, keepdims=True)
    pos_ov = jnp.max(ov * mp, axis=1, keepdims=True)
    norm = jnp.max(am * (pos_ov / (pos_align + _EPS)),
                   axis=0, keepdims=True)

    tb_ref[0] = _dot_exact(gb, oh_t, ((0,), (0,)),
                           split_lhs=True)
    cls_oh = jax.lax.dot_general(lab_oh, oh_t, (((0,), (0,)), ((), ())),
                                 preferred_element_type=f32,
                                 precision=jax.lax.Precision.DEFAULT)
    ts_ref[0] = cls_oh * jnp.where(fg > 0.0, norm, 0.0)
    fg_ref[0] = fg


@jax.jit
def kernel(pred_scores, pred_bboxes, anchor_points, gt_labels, gt_bboxes,
           mask_gt):
    bs, A, nc = pred_scores.shape
    n = gt_bboxes.shape[1]
    pst = jnp.transpose(pred_scores, (0, 2, 1))
    pbt = jnp.transpose(pred_bboxes, (0, 2, 1))
    apt = jnp.transpose(anchor_points, (1, 0))
    gl = gt_labels.astype(jnp.int32)
    mg = mask_gt.astype(jnp.float32)

    tb, ts, fg = pl.pallas_call(
        _assigner_body,
        grid=(bs,),
        in_specs=[
            pl.BlockSpec((1, nc, A), lambda b: (b, 0, 0)),
            pl.BlockSpec((1, 4, A), lambda b: (b, 0, 0)),
            pl.BlockSpec((2, A), lambda b: (0, 0)),
            pl.BlockSpec((1, n, 1), lambda b: (b, 0, 0)),
            pl.BlockSpec((1, n, 4), lambda b: (b, 0, 0)),
            pl.BlockSpec((1, n, 1), lambda b: (b, 0, 0)),
        ],
        out_specs=[
            pl.BlockSpec((1, 4, A), lambda b: (b, 0, 0)),
            pl.BlockSpec((1, nc, A), lambda b: (b, 0, 0)),
            pl.BlockSpec((1, 1, A), lambda b: (b, 0, 0)),
        ],
        out_shape=[
            jax.ShapeDtypeStruct((bs, 4, A), jnp.float32),
            jax.ShapeDtypeStruct((bs, nc, A), jnp.float32),
            jax.ShapeDtypeStruct((bs, 1, A), jnp.float32),
        ],
    )(pst.astype(jnp.float32), pbt, apt, gl,
      gt_bboxes.astype(jnp.float32), mg)

    return (jnp.transpose(tb, (0, 2, 1)), jnp.transpose(ts, (0, 2, 1)),
            fg.reshape(bs, A) > 0.0)
